# baseline (speedup 1.0000x reference)
import functools

import jax
import jax.numpy as jnp
from jax import lax
from jax.experimental import pallas as pl
from jax.experimental.pallas import tpu as pltpu
from jax.experimental.pallas import tpu_sc as plsc

_VOCAB = 100000
_DIM = 64
_B, _F = 4096, 26
_N = _B * _F
_NW = 32
_BPW = _B // _NW


def _body(idx_hbm, table_hbm, out_hbm, idx_v, idx_t, rows_a, rows_b,
          stage_a, stage_b, sem_i, sem_a, sem_b, sem_o):
    wid = lax.axis_index("s") * 2 + lax.axis_index("c")
    base = wid * (_BPW * _F)
    pltpu.sync_copy(idx_hbm.at[pl.ds(base, _BPW * _F)], idx_v)

    iota = lax.iota(jnp.int32, 16)
    iota26 = iota * _F
    for f in range(_F):
        for b in range(8):
            src = iota26 + (b * 16 * _F + f)
            idx_t[f, pl.ds(b * 16, 16)] = plsc.load_gather(idx_v, [src])

    def do_feature(f, rows_v, sem_g, stage):
        pltpu.async_copy(table_hbm.at[idx_t.at[f]], rows_v, sem_g).wait()
        for g in range(8):
            for r in range(8):
                col = jnp.full((16,), 8 * g + r, jnp.int32)
                for b in range(8):
                    v = plsc.load_gather(rows_v, [iota + b * 16, col])
                    stage[g, r, pl.ds(b * 16, 16)] = v
        copies = []
        for g in range(8):
            copies.append(
                pltpu.async_copy(stage.at[g], out_hbm.at[f, g, wid], sem_o)
            )
        for c in copies:
            c.wait()

    def loop_body(k, carry):
        do_feature(2 * k, rows_a, sem_a, stage_a)
        do_feature(2 * k + 1, rows_b, sem_b, stage_b)
        return carry

    lax.fori_loop(0, _F // 2, loop_body, 0)


_gather = pl.kernel(
    _body,
    mesh=plsc.VectorSubcoreMesh(core_axis_name="c", subcore_axis_name="s"),
    compiler_params=pltpu.CompilerParams(
        use_tc_tiling_on_sc=False, needs_layout_passes=False
    ),
    out_type=jax.ShapeDtypeStruct((_F, 8, _NW, 8, 128), jnp.float32),
    scratch_types=[
        pltpu.VMEM((_BPW * _F,), jnp.int32),
        pltpu.VMEM((_F, 128), jnp.int32),
        pltpu.VMEM((128, _DIM), jnp.float32),
        pltpu.VMEM((128, _DIM), jnp.float32),
        pltpu.VMEM((8, 8, 128), jnp.float32),
        pltpu.VMEM((8, 8, 128), jnp.float32),
        pltpu.SemaphoreType.DMA,
        pltpu.SemaphoreType.DMA,
        pltpu.SemaphoreType.DMA,
        pltpu.SemaphoreType.DMA,
    ],
)


@jax.jit
def kernel(ids, table):
    t5 = _gather(ids.reshape(_N), table)
    out = t5.transpose(2, 4, 0, 1, 3).reshape(_B, _F, _DIM)
    return out
